# Initial kernel scaffold; baseline (speedup 1.0000x reference)
import jax
import jax.numpy as jnp
from jax.experimental import pallas as pl


def kernel(x, edge_index, W1, b1, W2, b2, Wh, bh):
    raise NotImplementedError("write your pallas kernel here")


# speedup vs baseline: 10.7651x; 10.7651x over previous
import functools
import math

import jax
import jax.numpy as jnp
from jax import lax
from jax.experimental import pallas as pl
from jax.experimental.pallas import tpu as pltpu
from jax.experimental.pallas import tpu_sc as plsc

_NITER = 10
_CH = 32
_NTILES = 16
_KE = 400


def _make_edge_pass(N, E, interpret=False):
    EPT = E // _NTILES
    assert E % _NTILES == 0 and EPT % _KE == 0
    NCH = EPT // _KE
    RPT = (-(-N // _NTILES) + 7) // 8 * 8
    ACCN = RPT * _NTILES
    ZFULL, ZREM = divmod(RPT, _KE)
    mesh = plsc.VectorSubcoreMesh(core_axis_name="c", subcore_axis_name="s", num_cores=2, num_subcores=_NTILES)

    @functools.partial(
        pl.kernel,
        out_type=(jax.ShapeDtypeStruct((ACCN, _CH), jnp.float32),) * 2,
        mesh=mesh,
        scratch_types=[
            pltpu.VMEM((_KE,), jnp.int32),
            pltpu.VMEM((_KE,), jnp.int32),
            pltpu.VMEM((_KE, _CH), jnp.float32),
            pltpu.VMEM_SHARED((ACCN, _CH), jnp.float32),
            pltpu.SemaphoreType.DMA,
        ],
        compiler_params=pltpu.CompilerParams(use_tc_tiling_on_sc=False),
        interpret=interpret,
    )
    def edge_pass(u0_h, u1_h, src_h, dst_h, s0_h, s1_h,
                  sbuf, dbuf, rows, acc, sem):
        c = lax.axis_index("c")
        s = lax.axis_index("s")
        r0 = pl.multiple_of(s * RPT, 8)

        @pl.loop(0, _KE)
        def _zero(i):
            for h in range(_CH // 16):
                rows[i, pl.ds(h * 16, 16)] = jnp.zeros((16,), jnp.float32)

        for t in range(ZFULL):
            pltpu.sync_copy(rows, acc.at[pl.ds(r0 + t * _KE, _KE)])
        if ZREM:
            pltpu.sync_copy(rows.at[pl.ds(0, ZREM)],
                            acc.at[pl.ds(r0 + ZFULL * _KE, ZREM)])

        plsc.subcore_barrier()

        def phase(u_h, s_h):
            @pl.loop(0, NCH)
            def _edges(j):
                base = pl.multiple_of(s * EPT + j * _KE, 8)
                pltpu.sync_copy(src_h.at[pl.ds(base, _KE)], sbuf)
                pltpu.sync_copy(dst_h.at[pl.ds(base, _KE)], dbuf)
                pltpu.async_copy(u_h.at[sbuf], rows, sem).wait()
                pltpu.sync_copy(rows, acc.at[dbuf], add=True)

            plsc.subcore_barrier()
            for t in range(ZFULL):
                pltpu.sync_copy(acc.at[pl.ds(r0 + t * _KE, _KE)], rows)
                pltpu.sync_copy(rows, s_h.at[pl.ds(r0 + t * _KE, _KE)])
            if ZREM:
                pltpu.sync_copy(acc.at[pl.ds(r0 + ZFULL * _KE, ZREM)],
                                rows.at[pl.ds(0, ZREM)])
                pltpu.sync_copy(rows.at[pl.ds(0, ZREM)],
                                s_h.at[pl.ds(r0 + ZFULL * _KE, ZREM)])

        @pl.when(c == 0)
        def _():
            phase(u0_h, s0_h)

        @pl.when(c == 1)
        def _():
            phase(u1_h, s1_h)

    return edge_pass


_BR = 2000


def _prologue_body(x, W1, b1, W2, b2, dis,
                   lp_o, u0_o, u1_o):
    h = jnp.maximum(jnp.dot(x[...], W1[...],
                            preferred_element_type=jnp.float32) + b1[...], 0.0)
    lp = jnp.dot(h, W2[...], preferred_element_type=jnp.float32) + b2[...]
    lp_o[...] = lp
    d = dis[...]
    u0_o[...] = d * lp[:, :_CH]
    u1_o[...] = d * lp[:, _CH:]


def _make_prologue(N, D, C, interpret=False):
    grid = (N // _BR,)
    row = lambda i: (i, 0)
    return pl.pallas_call(
        _prologue_body,
        grid=grid,
        in_specs=[
            pl.BlockSpec((_BR, D), row),
            pl.BlockSpec((D, C), lambda i: (0, 0)),
            pl.BlockSpec((1, C), lambda i: (0, 0)),
            pl.BlockSpec((C, C), lambda i: (0, 0)),
            pl.BlockSpec((1, C), lambda i: (0, 0)),
            pl.BlockSpec((_BR, 1), row),
        ],
        out_specs=[
            pl.BlockSpec((_BR, C), row),
            pl.BlockSpec((_BR, _CH), row),
            pl.BlockSpec((_BR, _CH), row),
        ],
        out_shape=[
            jax.ShapeDtypeStruct((N, C), jnp.float32),
            jax.ShapeDtypeStruct((N, _CH), jnp.float32),
            jax.ShapeDtypeStruct((N, _CH), jnp.float32),
        ],
        interpret=interpret,
    )


def _mid_body(s0, s1, u0, u1, dis, dis2,
              prop_o, u0_o, u1_o):
    T0 = s0[...] + u0[...]
    T1 = s1[...] + u1[...]
    d = dis[...]
    prop_o[...] = jnp.concatenate([d * T0, d * T1], axis=1)
    d2 = dis2[...]
    u0_o[...] = d2 * T0
    u1_o[...] = d2 * T1


def _make_mid(N, C, interpret=False):
    grid = (N // _BR,)
    row = lambda i: (i, 0)
    half = pl.BlockSpec((_BR, _CH), row)
    col = pl.BlockSpec((_BR, 1), row)
    return pl.pallas_call(
        _mid_body,
        grid=grid,
        in_specs=[half, half, half, half, col, col],
        out_specs=[pl.BlockSpec((_BR, C), row), half, half],
        out_shape=[
            jax.ShapeDtypeStruct((N, C), jnp.float32),
            jax.ShapeDtypeStruct((N, _CH), jnp.float32),
            jax.ShapeDtypeStruct((N, _CH), jnp.float32),
        ],
        interpret=interpret,
    )


def _xacc_body(xacc, prop_new, prop_old, p, cont, xacc_o):
    pv = p[...]
    xacc_o[...] = xacc[...] + (pv * prop_new[...]
                               + (1.0 - pv) * prop_old[...]) * cont[...]


def _make_xacc(N, C, interpret=False):
    grid = (N // _BR,)
    row = lambda i: (i, 0)
    full = pl.BlockSpec((_BR, C), row)
    col = pl.BlockSpec((_BR, 1), row)
    return pl.pallas_call(
        _xacc_body,
        grid=grid,
        in_specs=[full, full, full, col, col],
        out_specs=full,
        out_shape=jax.ShapeDtypeStruct((N, C), jnp.float32),
        interpret=interpret,
    )


def _epilogue_body(xacc, steps, logp_o):
    xo = xacc[...] / steps[...]
    m = jnp.max(xo, axis=1, keepdims=True)
    e = jnp.exp(xo - m)
    lse = m + jnp.log(jnp.sum(e, axis=1, keepdims=True))
    logp_o[...] = xo - lse


def _make_epilogue(N, C, interpret=False):
    grid = (N // _BR,)
    row = lambda i: (i, 0)
    full = pl.BlockSpec((_BR, C), row)
    col = pl.BlockSpec((_BR, 1), row)
    return pl.pallas_call(
        _epilogue_body,
        grid=grid,
        in_specs=[full, col],
        out_specs=full,
        out_shape=jax.ShapeDtypeStruct((N, C), jnp.float32),
        interpret=interpret,
    )


def kernel(x, edge_index, W1, b1, W2, b2, Wh, bh):
    N, D = x.shape
    C = W2.shape[1]
    E = edge_index.shape[1]
    src = edge_index[0]
    dst = edge_index[1]

    edge_pass = _make_edge_pass(N, E)

    onesu = jnp.ones((N, _CH), jnp.float32)
    c0, _ = edge_pass(onesu, onesu, src, dst)
    deg = c0[:N, 0] + 1.0
    dis = deg ** -0.5
    dis2 = dis * dis
    disc = dis.reshape(N, 1)
    dis2c = dis2.reshape(N, 1)

    prop, u0, u1 = _make_prologue(N, D, C)(
        x, W1, b1.reshape(1, C), W2, b2.reshape(1, C), disc)

    mid = _make_mid(N, C)
    xacc_k = _make_xacc(N, C)
    steps = jnp.ones((N,), jnp.float32)
    sum_h = jnp.zeros((N,), jnp.float32)
    continue_mask = jnp.ones((N,), bool)
    x_acc = jnp.zeros((N, C), jnp.float32)

    for i in range(_NITER):
        s0, s1 = edge_pass(u0, u1, src, dst)
        prop_new, u0, u1 = mid(s0, s1, u0, u1, disc, dis2c)
        hh = jax.nn.sigmoid(prop_new @ Wh + bh).squeeze(-1)
        prob_mask = ((sum_h + hh) < 0.99) & continue_mask
        prob_fmask = prob_mask.astype(jnp.float32)
        if i == _NITER - 1:
            last_iteration_mask = jnp.zeros((N,), dtype=jnp.float32)
        else:
            last_iteration_mask = jnp.ones((N,), dtype=jnp.float32)
        steps = steps + prob_fmask * last_iteration_mask
        sum_h = sum_h + prob_fmask * hh
        final_iter = steps < _NITER
        condition = prob_mask & final_iter
        p = jnp.where(condition, sum_h, 1.0 - sum_h)
        x_acc = xacc_k(x_acc, prop_new, prop,
                       p.reshape(N, 1),
                       continue_mask.astype(jnp.float32).reshape(N, 1))
        continue_mask = continue_mask & prob_mask
        prop = prop_new

    logp = _make_epilogue(N, C)(x_acc, steps.reshape(N, 1))
    return (logp, steps, 1.0 - sum_h)


# speedup vs baseline: 13.8690x; 1.2883x over previous
import functools
import math

import jax
import jax.numpy as jnp
from jax import lax
from jax.experimental import pallas as pl
from jax.experimental.pallas import tpu as pltpu
from jax.experimental.pallas import tpu_sc as plsc

_NITER = 10
_CH = 32
_NTILES = 16
_KE = 400


def _make_edge_pass(N, E, interpret=False):
    EPT = E // _NTILES
    assert E % _NTILES == 0 and EPT % _KE == 0
    NCH = EPT // _KE
    RPT = (-(-N // _NTILES) + 7) // 8 * 8
    ACCN = RPT * _NTILES
    ZFULL, ZREM = divmod(RPT, _KE)
    mesh = plsc.VectorSubcoreMesh(core_axis_name="c", subcore_axis_name="s", num_cores=2, num_subcores=_NTILES)

    @functools.partial(
        pl.kernel,
        out_type=(jax.ShapeDtypeStruct((ACCN, _CH), jnp.float32),) * 2,
        mesh=mesh,
        scratch_types=[
            pltpu.VMEM((2, _KE), jnp.int32),
            pltpu.VMEM((2, _KE), jnp.int32),
            pltpu.VMEM((2, _KE, _CH), jnp.float32),
            pltpu.VMEM_SHARED((ACCN, _CH), jnp.float32),
            pltpu.SemaphoreType.DMA,
            pltpu.SemaphoreType.DMA,
            pltpu.SemaphoreType.DMA,
            pltpu.SemaphoreType.DMA,
        ],
        compiler_params=pltpu.CompilerParams(use_tc_tiling_on_sc=False),
        interpret=interpret,
    )
    def edge_pass(u0_h, u1_h, src_h, dst_h, s0_h, s1_h,
                  sbuf, dbuf, rows, acc, gsem0, gsem1, ssem0, ssem1):
        c = lax.axis_index("c")
        s = lax.axis_index("s")
        r0 = pl.multiple_of(s * RPT, 8)
        gsem = (gsem0, gsem1)
        ssem = (ssem0, ssem1)
        zrows = rows.at[0]

        @pl.loop(0, _KE)
        def _zero(i):
            for h in range(_CH // 16):
                rows[0, i, pl.ds(h * 16, 16)] = jnp.zeros((16,), jnp.float32)

        for t in range(ZFULL):
            pltpu.sync_copy(zrows, acc.at[pl.ds(r0 + t * _KE, _KE)])
        if ZREM:
            pltpu.sync_copy(zrows.at[pl.ds(0, ZREM)],
                            acc.at[pl.ds(r0 + ZFULL * _KE, ZREM)])

        plsc.subcore_barrier()

        def phase(u_h, s_h):
            def chunk_base(j):
                return pl.multiple_of(s * EPT + j * _KE, 8)

            def load_idx(j, b):
                base = chunk_base(j)
                pltpu.sync_copy(src_h.at[pl.ds(base, _KE)], sbuf.at[b])
                pltpu.sync_copy(dst_h.at[pl.ds(base, _KE)], dbuf.at[b])

            def start_gather(b):
                pltpu.async_copy(u_h.at[sbuf.at[b]], rows.at[b], gsem[b])

            def wait_gather(b):
                pltpu.make_async_copy(u_h.at[sbuf.at[b]], rows.at[b],
                                      gsem[b]).wait()

            def start_scatter(b):
                pltpu.async_copy(rows.at[b], acc.at[dbuf.at[b]], ssem[b],
                                 add=True)

            def wait_scatter(b):
                pltpu.make_async_copy(rows.at[b], acc.at[dbuf.at[b]],
                                      ssem[b]).wait()

            for b in range(2):
                load_idx(b, b)
                start_gather(b)

            NPAIR = (NCH - 1) // 2

            @pl.loop(0, NPAIR)
            def _pairs(jo):
                for b in range(2):
                    j = jo * 2 + b
                    wait_gather(b)
                    start_scatter(b)
                    jn = j + 2

                    @pl.when(jn < NCH)
                    def _():
                        wait_scatter(b)
                        load_idx(jn, b)
                        start_gather(b)

            for j in range(2 * NPAIR, NCH):
                b = j % 2
                wait_gather(b)
                start_scatter(b)
            wait_scatter((NCH - 2) % 2)
            wait_scatter((NCH - 1) % 2)

            plsc.subcore_barrier()
            for t in range(ZFULL):
                pltpu.sync_copy(acc.at[pl.ds(r0 + t * _KE, _KE)], zrows)
                pltpu.sync_copy(zrows, s_h.at[pl.ds(r0 + t * _KE, _KE)])
            if ZREM:
                pltpu.sync_copy(acc.at[pl.ds(r0 + ZFULL * _KE, ZREM)],
                                zrows.at[pl.ds(0, ZREM)])
                pltpu.sync_copy(zrows.at[pl.ds(0, ZREM)],
                                s_h.at[pl.ds(r0 + ZFULL * _KE, ZREM)])

        @pl.when(c == 0)
        def _():
            phase(u0_h, s0_h)

        @pl.when(c == 1)
        def _():
            phase(u1_h, s1_h)

    return edge_pass


_BR = 2000


def _prologue_body(x, W1, b1, W2, b2, dis,
                   lp_o, u0_o, u1_o):
    h = jnp.maximum(jnp.dot(x[...], W1[...],
                            preferred_element_type=jnp.float32) + b1[...], 0.0)
    lp = jnp.dot(h, W2[...], preferred_element_type=jnp.float32) + b2[...]
    lp_o[...] = lp
    d = dis[...]
    u0_o[...] = d * lp[:, :_CH]
    u1_o[...] = d * lp[:, _CH:]


def _make_prologue(N, D, C, interpret=False):
    grid = (N // _BR,)
    row = lambda i: (i, 0)
    return pl.pallas_call(
        _prologue_body,
        grid=grid,
        in_specs=[
            pl.BlockSpec((_BR, D), row),
            pl.BlockSpec((D, C), lambda i: (0, 0)),
            pl.BlockSpec((1, C), lambda i: (0, 0)),
            pl.BlockSpec((C, C), lambda i: (0, 0)),
            pl.BlockSpec((1, C), lambda i: (0, 0)),
            pl.BlockSpec((_BR, 1), row),
        ],
        out_specs=[
            pl.BlockSpec((_BR, C), row),
            pl.BlockSpec((_BR, _CH), row),
            pl.BlockSpec((_BR, _CH), row),
        ],
        out_shape=[
            jax.ShapeDtypeStruct((N, C), jnp.float32),
            jax.ShapeDtypeStruct((N, _CH), jnp.float32),
            jax.ShapeDtypeStruct((N, _CH), jnp.float32),
        ],
        interpret=interpret,
    )


def _mid_body(s0, s1, u0, u1, dis, dis2,
              prop_o, u0_o, u1_o):
    T0 = s0[...] + u0[...]
    T1 = s1[...] + u1[...]
    d = dis[...]
    prop_o[...] = jnp.concatenate([d * T0, d * T1], axis=1)
    d2 = dis2[...]
    u0_o[...] = d2 * T0
    u1_o[...] = d2 * T1


def _make_mid(N, C, interpret=False):
    grid = (N // _BR,)
    row = lambda i: (i, 0)
    half = pl.BlockSpec((_BR, _CH), row)
    col = pl.BlockSpec((_BR, 1), row)
    return pl.pallas_call(
        _mid_body,
        grid=grid,
        in_specs=[half, half, half, half, col, col],
        out_specs=[pl.BlockSpec((_BR, C), row), half, half],
        out_shape=[
            jax.ShapeDtypeStruct((N, C), jnp.float32),
            jax.ShapeDtypeStruct((N, _CH), jnp.float32),
            jax.ShapeDtypeStruct((N, _CH), jnp.float32),
        ],
        interpret=interpret,
    )


def _xacc_body(xacc, prop_new, prop_old, p, cont, xacc_o):
    pv = p[...]
    xacc_o[...] = xacc[...] + (pv * prop_new[...]
                               + (1.0 - pv) * prop_old[...]) * cont[...]


def _make_xacc(N, C, interpret=False):
    grid = (N // _BR,)
    row = lambda i: (i, 0)
    full = pl.BlockSpec((_BR, C), row)
    col = pl.BlockSpec((_BR, 1), row)
    return pl.pallas_call(
        _xacc_body,
        grid=grid,
        in_specs=[full, full, full, col, col],
        out_specs=full,
        out_shape=jax.ShapeDtypeStruct((N, C), jnp.float32),
        interpret=interpret,
    )


def _epilogue_body(xacc, steps, logp_o):
    xo = xacc[...] / steps[...]
    m = jnp.max(xo, axis=1, keepdims=True)
    e = jnp.exp(xo - m)
    lse = m + jnp.log(jnp.sum(e, axis=1, keepdims=True))
    logp_o[...] = xo - lse


def _make_epilogue(N, C, interpret=False):
    grid = (N // _BR,)
    row = lambda i: (i, 0)
    full = pl.BlockSpec((_BR, C), row)
    col = pl.BlockSpec((_BR, 1), row)
    return pl.pallas_call(
        _epilogue_body,
        grid=grid,
        in_specs=[full, col],
        out_specs=full,
        out_shape=jax.ShapeDtypeStruct((N, C), jnp.float32),
        interpret=interpret,
    )


def kernel(x, edge_index, W1, b1, W2, b2, Wh, bh):
    N, D = x.shape
    C = W2.shape[1]
    E = edge_index.shape[1]
    src = edge_index[0]
    dst = edge_index[1]

    edge_pass = _make_edge_pass(N, E)

    onesu = jnp.ones((N, _CH), jnp.float32)
    c0, _ = edge_pass(onesu, onesu, src, dst)
    deg = c0[:N, 0] + 1.0
    dis = deg ** -0.5
    dis2 = dis * dis
    disc = dis.reshape(N, 1)
    dis2c = dis2.reshape(N, 1)

    prop, u0, u1 = _make_prologue(N, D, C)(
        x, W1, b1.reshape(1, C), W2, b2.reshape(1, C), disc)

    mid = _make_mid(N, C)
    xacc_k = _make_xacc(N, C)
    steps = jnp.ones((N,), jnp.float32)
    sum_h = jnp.zeros((N,), jnp.float32)
    continue_mask = jnp.ones((N,), bool)
    x_acc = jnp.zeros((N, C), jnp.float32)

    for i in range(_NITER):
        s0, s1 = edge_pass(u0, u1, src, dst)
        prop_new, u0, u1 = mid(s0, s1, u0, u1, disc, dis2c)
        hh = jax.nn.sigmoid(prop_new @ Wh + bh).squeeze(-1)
        prob_mask = ((sum_h + hh) < 0.99) & continue_mask
        prob_fmask = prob_mask.astype(jnp.float32)
        if i == _NITER - 1:
            last_iteration_mask = jnp.zeros((N,), dtype=jnp.float32)
        else:
            last_iteration_mask = jnp.ones((N,), dtype=jnp.float32)
        steps = steps + prob_fmask * last_iteration_mask
        sum_h = sum_h + prob_fmask * hh
        final_iter = steps < _NITER
        condition = prob_mask & final_iter
        p = jnp.where(condition, sum_h, 1.0 - sum_h)
        x_acc = xacc_k(x_acc, prop_new, prop,
                       p.reshape(N, 1),
                       continue_mask.astype(jnp.float32).reshape(N, 1))
        continue_mask = continue_mask & prob_mask
        prop = prop_new

    logp = _make_epilogue(N, C)(x_acc, steps.reshape(N, 1))
    return (logp, steps, 1.0 - sum_h)


# speedup vs baseline: 15.4804x; 1.1162x over previous
import functools
import math

import jax
import jax.numpy as jnp
from jax import lax
from jax.experimental import pallas as pl
from jax.experimental.pallas import tpu as pltpu
from jax.experimental.pallas import tpu_sc as plsc

_NITER = 10
_CH = 32
_NTILES = 16
_KE = 400


def _make_edge_pass(N, E, interpret=False):
    EPT = E // _NTILES
    assert E % _NTILES == 0 and EPT % _KE == 0
    NCH = EPT // _KE
    RPT = (-(-N // _NTILES) + 7) // 8 * 8
    ACCN = RPT * _NTILES
    ZFULL, ZREM = divmod(RPT, _KE)
    mesh = plsc.VectorSubcoreMesh(core_axis_name="c", subcore_axis_name="s", num_cores=2, num_subcores=_NTILES)

    @functools.partial(
        pl.kernel,
        out_type=(jax.ShapeDtypeStruct((ACCN, _CH), jnp.float32),) * 2,
        mesh=mesh,
        scratch_types=[
            pltpu.VMEM((2, _KE), jnp.int32),
            pltpu.VMEM((2, _KE), jnp.int32),
            pltpu.VMEM((2, _KE, _CH), jnp.float32),
            pltpu.VMEM_SHARED((ACCN, _CH), jnp.float32),
            pltpu.SemaphoreType.DMA,
            pltpu.SemaphoreType.DMA,
            pltpu.SemaphoreType.DMA,
            pltpu.SemaphoreType.DMA,
            pltpu.SemaphoreType.DMA,
        ],
        compiler_params=pltpu.CompilerParams(use_tc_tiling_on_sc=False),
        interpret=interpret,
    )
    def edge_pass(u0_h, u1_h, src_h, dst_h, s0_h, s1_h,
                  sbuf, dbuf, rows, acc, gsem0, gsem1, ssem0, ssem1, isem):
        c = lax.axis_index("c")
        s = lax.axis_index("s")
        r0 = pl.multiple_of(s * RPT, 8)
        gsem = (gsem0, gsem1)
        ssem = (ssem0, ssem1)
        zrows = rows.at[0]

        @pl.loop(0, _KE)
        def _zero(i):
            for h in range(_CH // 16):
                rows[0, i, pl.ds(h * 16, 16)] = jnp.zeros((16,), jnp.float32)

        for t in range(ZFULL):
            pltpu.sync_copy(zrows, acc.at[pl.ds(r0 + t * _KE, _KE)])
        if ZREM:
            pltpu.sync_copy(zrows.at[pl.ds(0, ZREM)],
                            acc.at[pl.ds(r0 + ZFULL * _KE, ZREM)])

        plsc.subcore_barrier()

        def phase(u_h, s_h):
            def chunk_base(j):
                return pl.multiple_of(s * EPT + j * _KE, 8)

            def load_idx(j, b):
                base = chunk_base(j)
                pltpu.async_copy(src_h.at[pl.ds(base, _KE)], sbuf.at[b], isem)
                pltpu.async_copy(dst_h.at[pl.ds(base, _KE)], dbuf.at[b], isem)
                pltpu.make_async_copy(src_h.at[pl.ds(base, _KE)],
                                      sbuf.at[b], isem).wait()
                pltpu.make_async_copy(dst_h.at[pl.ds(base, _KE)],
                                      dbuf.at[b], isem).wait()

            def start_gather(b):
                pltpu.async_copy(u_h.at[sbuf.at[b]], rows.at[b], gsem[b])

            def wait_gather(b):
                pltpu.make_async_copy(u_h.at[sbuf.at[b]], rows.at[b],
                                      gsem[b]).wait()

            def start_scatter(b):
                pltpu.async_copy(rows.at[b], acc.at[dbuf.at[b]], ssem[b],
                                 add=True)

            def wait_scatter(b):
                pltpu.make_async_copy(rows.at[b], acc.at[dbuf.at[b]],
                                      ssem[b]).wait()

            for b in range(2):
                load_idx(b, b)
                start_gather(b)

            NPAIR = (NCH - 1) // 2

            @pl.loop(0, NPAIR)
            def _pairs(jo):
                for b in range(2):
                    j = jo * 2 + b
                    wait_gather(b)
                    start_scatter(b)
                    jn = j + 2

                    @pl.when(jn < NCH)
                    def _():
                        wait_scatter(b)
                        load_idx(jn, b)
                        start_gather(b)

            for j in range(2 * NPAIR, NCH):
                b = j % 2
                wait_gather(b)
                start_scatter(b)
            wait_scatter((NCH - 2) % 2)
            wait_scatter((NCH - 1) % 2)

            plsc.subcore_barrier()
            for t in range(ZFULL):
                pltpu.sync_copy(acc.at[pl.ds(r0 + t * _KE, _KE)], zrows)
                pltpu.sync_copy(zrows, s_h.at[pl.ds(r0 + t * _KE, _KE)])
            if ZREM:
                pltpu.sync_copy(acc.at[pl.ds(r0 + ZFULL * _KE, ZREM)],
                                zrows.at[pl.ds(0, ZREM)])
                pltpu.sync_copy(zrows.at[pl.ds(0, ZREM)],
                                s_h.at[pl.ds(r0 + ZFULL * _KE, ZREM)])

        @pl.when(c == 0)
        def _():
            phase(u0_h, s0_h)

        @pl.when(c == 1)
        def _():
            phase(u1_h, s1_h)

    return edge_pass


_BR = 2000


def _prologue_body(x, W1, b1, W2, b2, dis,
                   lp_o, u0_o, u1_o):
    h = jnp.maximum(jnp.dot(x[...], W1[...],
                            preferred_element_type=jnp.float32) + b1[...], 0.0)
    lp = jnp.dot(h, W2[...], preferred_element_type=jnp.float32) + b2[...]
    lp_o[...] = lp
    d = dis[...]
    u0_o[...] = d * lp[:, :_CH]
    u1_o[...] = d * lp[:, _CH:]


def _make_prologue(N, D, C, interpret=False):
    grid = (N // _BR,)
    row = lambda i: (i, 0)
    return pl.pallas_call(
        _prologue_body,
        grid=grid,
        in_specs=[
            pl.BlockSpec((_BR, D), row),
            pl.BlockSpec((D, C), lambda i: (0, 0)),
            pl.BlockSpec((1, C), lambda i: (0, 0)),
            pl.BlockSpec((C, C), lambda i: (0, 0)),
            pl.BlockSpec((1, C), lambda i: (0, 0)),
            pl.BlockSpec((_BR, 1), row),
        ],
        out_specs=[
            pl.BlockSpec((_BR, C), row),
            pl.BlockSpec((_BR, _CH), row),
            pl.BlockSpec((_BR, _CH), row),
        ],
        out_shape=[
            jax.ShapeDtypeStruct((N, C), jnp.float32),
            jax.ShapeDtypeStruct((N, _CH), jnp.float32),
            jax.ShapeDtypeStruct((N, _CH), jnp.float32),
        ],
        interpret=interpret,
    )


def _mid_body(s0, s1, u0, u1, dis, dis2,
              prop_o, u0_o, u1_o):
    T0 = s0[...] + u0[...]
    T1 = s1[...] + u1[...]
    d = dis[...]
    prop_o[...] = jnp.concatenate([d * T0, d * T1], axis=1)
    d2 = dis2[...]
    u0_o[...] = d2 * T0
    u1_o[...] = d2 * T1


def _make_mid(N, C, interpret=False):
    grid = (N // _BR,)
    row = lambda i: (i, 0)
    half = pl.BlockSpec((_BR, _CH), row)
    col = pl.BlockSpec((_BR, 1), row)
    return pl.pallas_call(
        _mid_body,
        grid=grid,
        in_specs=[half, half, half, half, col, col],
        out_specs=[pl.BlockSpec((_BR, C), row), half, half],
        out_shape=[
            jax.ShapeDtypeStruct((N, C), jnp.float32),
            jax.ShapeDtypeStruct((N, _CH), jnp.float32),
            jax.ShapeDtypeStruct((N, _CH), jnp.float32),
        ],
        interpret=interpret,
    )


def _xacc_body(xacc, prop_new, prop_old, p, cont, xacc_o):
    pv = p[...]
    xacc_o[...] = xacc[...] + (pv * prop_new[...]
                               + (1.0 - pv) * prop_old[...]) * cont[...]


def _make_xacc(N, C, interpret=False):
    grid = (N // _BR,)
    row = lambda i: (i, 0)
    full = pl.BlockSpec((_BR, C), row)
    col = pl.BlockSpec((_BR, 1), row)
    return pl.pallas_call(
        _xacc_body,
        grid=grid,
        in_specs=[full, full, full, col, col],
        out_specs=full,
        out_shape=jax.ShapeDtypeStruct((N, C), jnp.float32),
        interpret=interpret,
    )


def _epilogue_body(xacc, steps, logp_o):
    xo = xacc[...] / steps[...]
    m = jnp.max(xo, axis=1, keepdims=True)
    e = jnp.exp(xo - m)
    lse = m + jnp.log(jnp.sum(e, axis=1, keepdims=True))
    logp_o[...] = xo - lse


def _make_epilogue(N, C, interpret=False):
    grid = (N // _BR,)
    row = lambda i: (i, 0)
    full = pl.BlockSpec((_BR, C), row)
    col = pl.BlockSpec((_BR, 1), row)
    return pl.pallas_call(
        _epilogue_body,
        grid=grid,
        in_specs=[full, col],
        out_specs=full,
        out_shape=jax.ShapeDtypeStruct((N, C), jnp.float32),
        interpret=interpret,
    )


def kernel(x, edge_index, W1, b1, W2, b2, Wh, bh):
    N, D = x.shape
    C = W2.shape[1]
    E = edge_index.shape[1]
    src = edge_index[0]
    dst = edge_index[1]

    edge_pass = _make_edge_pass(N, E)

    onesu = jnp.ones((N, _CH), jnp.float32)
    c0, _ = edge_pass(onesu, onesu, src, dst)
    deg = c0[:N, 0] + 1.0
    dis = deg ** -0.5
    dis2 = dis * dis
    disc = dis.reshape(N, 1)
    dis2c = dis2.reshape(N, 1)

    prop, u0, u1 = _make_prologue(N, D, C)(
        x, W1, b1.reshape(1, C), W2, b2.reshape(1, C), disc)

    mid = _make_mid(N, C)
    xacc_k = _make_xacc(N, C)
    steps = jnp.ones((N,), jnp.float32)
    sum_h = jnp.zeros((N,), jnp.float32)
    continue_mask = jnp.ones((N,), bool)
    x_acc = jnp.zeros((N, C), jnp.float32)

    for i in range(_NITER):
        s0, s1 = edge_pass(u0, u1, src, dst)
        prop_new, u0, u1 = mid(s0, s1, u0, u1, disc, dis2c)
        hh = jax.nn.sigmoid(prop_new @ Wh + bh).squeeze(-1)
        prob_mask = ((sum_h + hh) < 0.99) & continue_mask
        prob_fmask = prob_mask.astype(jnp.float32)
        if i == _NITER - 1:
            last_iteration_mask = jnp.zeros((N,), dtype=jnp.float32)
        else:
            last_iteration_mask = jnp.ones((N,), dtype=jnp.float32)
        steps = steps + prob_fmask * last_iteration_mask
        sum_h = sum_h + prob_fmask * hh
        final_iter = steps < _NITER
        condition = prob_mask & final_iter
        p = jnp.where(condition, sum_h, 1.0 - sum_h)
        x_acc = xacc_k(x_acc, prop_new, prop,
                       p.reshape(N, 1),
                       continue_mask.astype(jnp.float32).reshape(N, 1))
        continue_mask = continue_mask & prob_mask
        prop = prop_new

    logp = _make_epilogue(N, C)(x_acc, steps.reshape(N, 1))
    return (logp, steps, 1.0 - sum_h)
